# Initial kernel scaffold; baseline (speedup 1.0000x reference)
import jax
import jax.numpy as jnp
from jax.experimental import pallas as pl


def kernel(x_categories, x_numeric, tables, bn_num_g, bn_num_b, W1, b1, g1, be1, W2, b2, g2, be2, W3, b3):
    raise NotImplementedError("write your pallas kernel here")


# speedup vs baseline: 7.5086x; 7.5086x over previous
import functools

import jax
import jax.numpy as jnp
from jax import lax
from jax.experimental import pallas as pl
from jax.experimental.pallas import tpu as pltpu
from jax.experimental.pallas import tpu_sc as plsc

B = 16384
F = 26
V = 100000
D = 16
NUM = 13
H1 = 512
H2 = 256
EMB = F * D
EPS = 1e-5

NC = 2
NS = 16
NW = NC * NS
ROWS = B * F
RPW = ROWS // NW
CHUNK = 128
NCH = RPW // CHUNK
SUP = 8
NSUP = NCH // SUP
BUFROWS = SUP * CHUNK

RT = 1024
NT = B // RT


def _sc_gather(tab_flat, idx2d):
    mesh = plsc.VectorSubcoreMesh(core_axis_name="c", subcore_axis_name="s")

    @functools.partial(
        pl.kernel,
        out_type=jax.ShapeDtypeStruct((ROWS, D), jnp.float32),
        mesh=mesh,
        scratch_types=[
            pltpu.VMEM((NCH, CHUNK), jnp.int32),
            pltpu.VMEM((BUFROWS, D), jnp.float32),
            pltpu.VMEM((BUFROWS, D), jnp.float32),
            pltpu.SemaphoreType.DMA,
            pltpu.SemaphoreType.DMA,
            pltpu.SemaphoreType.DMA,
        ],
        compiler_params=pltpu.CompilerParams(use_tc_tiling_on_sc=False),
    )
    def gather_k(tab_hbm, idx_hbm, out_hbm, idx_v, buf0, buf1, gsem, ssem0, ssem1):
        wid = lax.axis_index("s") * NC + lax.axis_index("c")
        pltpu.sync_copy(idx_hbm.at[pl.ds(wid * NCH, NCH)], idx_v)
        bufs = (buf0, buf1)
        ssems = (ssem0, ssem1)
        store = [None, None]
        for g in range(NSUP):
            bi = g % 2
            buf = bufs[bi]
            if store[bi] is not None:
                store[bi].wait()
            fires = [
                pltpu.async_copy(
                    tab_hbm.at[idx_v.at[g * SUP + i]],
                    buf.at[pl.ds(i * CHUNK, CHUNK)],
                    gsem,
                )
                for i in range(SUP)
            ]
            for h in fires:
                h.wait()
            store[bi] = pltpu.async_copy(
                buf, out_hbm.at[pl.ds(wid * RPW + g * BUFROWS, BUFROWS)], ssems[bi]
            )
        for h in store:
            if h is not None:
                h.wait()

    return gather_k(tab_flat, idx2d)


def _numstats_body(xn_ref, out_ref):
    i = pl.program_id(0)
    x = xn_ref[...]
    sq = jnp.concatenate(
        [jnp.sum(x, axis=0, keepdims=True), jnp.sum(x * x, axis=0, keepdims=True)],
        axis=0,
    )

    @pl.when(i == 0)
    def _():
        out_ref[...] = sq

    @pl.when(i > 0)
    def _():
        out_ref[...] += sq


def _numstats(xn):
    return pl.pallas_call(
        _numstats_body,
        grid=(NT,),
        in_specs=[pl.BlockSpec((RT, NUM), lambda i: (i, 0))],
        out_specs=pl.BlockSpec((2, NUM), lambda i: (0, 0)),
        out_shape=jax.ShapeDtypeStruct((2, NUM), jnp.float32),
    )(xn)


def _pass1_body(xe_ref, xn_ref, w1eT_ref, w1nT_ref, b1_ref, gn_ref, bn_ref,
                sn_ref, h1_ref, st1_ref):
    i = pl.program_id(0)
    mn = sn_ref[0:1, :] * (1.0 / B)
    varn = sn_ref[1:2, :] * (1.0 / B) - mn * mn
    an = gn_ref[...] * lax.rsqrt(varn + EPS)
    cn = bn_ref[...] - mn * an
    xnn = xn_ref[...] * an + cn
    z = jnp.dot(xe_ref[...], w1eT_ref[...], preferred_element_type=jnp.float32)
    z = z + jnp.dot(xnn, w1nT_ref[...], preferred_element_type=jnp.float32)
    z = z + b1_ref[...]
    h = jnp.maximum(z, 0.0)
    h1_ref[...] = h
    sq = jnp.concatenate(
        [jnp.sum(h, axis=0, keepdims=True), jnp.sum(h * h, axis=0, keepdims=True)],
        axis=0,
    )

    @pl.when(i == 0)
    def _():
        st1_ref[...] = sq

    @pl.when(i > 0)
    def _():
        st1_ref[...] += sq


def _pass1(x_emb, xn, w1eT, w1nT, b1r, gnr, bnr, stats_n):
    return pl.pallas_call(
        _pass1_body,
        grid=(NT,),
        in_specs=[
            pl.BlockSpec((RT, EMB), lambda i: (i, 0)),
            pl.BlockSpec((RT, NUM), lambda i: (i, 0)),
            pl.BlockSpec((EMB, H1), lambda i: (0, 0)),
            pl.BlockSpec((NUM, H1), lambda i: (0, 0)),
            pl.BlockSpec((1, H1), lambda i: (0, 0)),
            pl.BlockSpec((1, NUM), lambda i: (0, 0)),
            pl.BlockSpec((1, NUM), lambda i: (0, 0)),
            pl.BlockSpec((2, NUM), lambda i: (0, 0)),
        ],
        out_specs=[
            pl.BlockSpec((RT, H1), lambda i: (i, 0)),
            pl.BlockSpec((2, H1), lambda i: (0, 0)),
        ],
        out_shape=[
            jax.ShapeDtypeStruct((B, H1), jnp.float32),
            jax.ShapeDtypeStruct((2, H1), jnp.float32),
        ],
    )(x_emb, xn, w1eT, w1nT, b1r, gnr, bnr, stats_n)


def _pass2_body(h1_ref, w2T_ref, b2_ref, g1_ref, be1_ref, st1_ref,
                h2_ref, st2_ref):
    i = pl.program_id(0)
    m1 = st1_ref[0:1, :] * (1.0 / B)
    var1 = st1_ref[1:2, :] * (1.0 / B) - m1 * m1
    a1 = g1_ref[...] * lax.rsqrt(var1 + EPS)
    c1 = be1_ref[...] - m1 * a1
    w2T = w2T_ref[...]
    w2eff = w2T * a1.reshape(H1, 1)
    bias = b2_ref[...] + jnp.dot(c1, w2T, preferred_element_type=jnp.float32)
    z = jnp.dot(h1_ref[...], w2eff, preferred_element_type=jnp.float32) + bias
    h = jnp.maximum(z, 0.0)
    h2_ref[...] = h
    sq = jnp.concatenate(
        [jnp.sum(h, axis=0, keepdims=True), jnp.sum(h * h, axis=0, keepdims=True)],
        axis=0,
    )

    @pl.when(i == 0)
    def _():
        st2_ref[...] = sq

    @pl.when(i > 0)
    def _():
        st2_ref[...] += sq


def _pass2(h1, w2T, b2r, g1r, be1r, stats1):
    return pl.pallas_call(
        _pass2_body,
        grid=(NT,),
        in_specs=[
            pl.BlockSpec((RT, H1), lambda i: (i, 0)),
            pl.BlockSpec((H1, H2), lambda i: (0, 0)),
            pl.BlockSpec((1, H2), lambda i: (0, 0)),
            pl.BlockSpec((1, H1), lambda i: (0, 0)),
            pl.BlockSpec((1, H1), lambda i: (0, 0)),
            pl.BlockSpec((2, H1), lambda i: (0, 0)),
        ],
        out_specs=[
            pl.BlockSpec((RT, H2), lambda i: (i, 0)),
            pl.BlockSpec((2, H2), lambda i: (0, 0)),
        ],
        out_shape=[
            jax.ShapeDtypeStruct((B, H2), jnp.float32),
            jax.ShapeDtypeStruct((2, H2), jnp.float32),
        ],
    )(h1, w2T, b2r, g1r, be1r, stats1)


def _pass3_body(h2_ref, w3_ref, b3_ref, g2_ref, be2_ref, st2_ref, out_ref):
    m2 = st2_ref[0:1, :] * (1.0 / B)
    var2 = st2_ref[1:2, :] * (1.0 / B) - m2 * m2
    a2 = g2_ref[...] * lax.rsqrt(var2 + EPS)
    c2 = be2_ref[...] - m2 * a2
    w3 = w3_ref[...]
    weff = w3 * a2
    bias = b3_ref[0, 0] + jnp.sum(c2 * w3)
    out_ref[...] = jnp.sum(h2_ref[...] * weff, axis=1, keepdims=True) + bias


def _pass3(h2, w3, b3r, g2r, be2r, stats2):
    return pl.pallas_call(
        _pass3_body,
        grid=(NT,),
        in_specs=[
            pl.BlockSpec((RT, H2), lambda i: (i, 0)),
            pl.BlockSpec((1, H2), lambda i: (0, 0)),
            pl.BlockSpec((1, 1), lambda i: (0, 0)),
            pl.BlockSpec((1, H2), lambda i: (0, 0)),
            pl.BlockSpec((1, H2), lambda i: (0, 0)),
            pl.BlockSpec((2, H2), lambda i: (0, 0)),
        ],
        out_specs=pl.BlockSpec((RT, 1), lambda i: (i, 0)),
        out_shape=jax.ShapeDtypeStruct((B, 1), jnp.float32),
    )(h2, w3, b3r, g2r, be2r, stats2)


def kernel(x_categories, x_numeric, tables, bn_num_g, bn_num_b,
           W1, b1, g1, be1, W2, b2, g2, be2, W3, b3):
    tab_flat = tables.reshape(F * V, D)
    flat_idx = (
        x_categories.astype(jnp.int32)
        + (jnp.arange(F, dtype=jnp.int32) * V)[None, :]
    ).reshape(ROWS // CHUNK, CHUNK)

    x_emb = _sc_gather(tab_flat, flat_idx).reshape(B, EMB)
    stats_n = _numstats(x_numeric)

    w1T = W1.T
    h1, stats1 = _pass1(
        x_emb, x_numeric,
        w1T[:EMB, :], w1T[EMB:, :],
        b1.reshape(1, H1), bn_num_g.reshape(1, NUM), bn_num_b.reshape(1, NUM),
        stats_n,
    )
    h2, stats2 = _pass2(
        h1, W2.T, b2.reshape(1, H2), g1.reshape(1, H1), be1.reshape(1, H1), stats1
    )
    out = _pass3(
        h2, W3, b3.reshape(1, 1), g2.reshape(1, H2), be2.reshape(1, H2), stats2
    )
    return out
